# baseline (speedup 1.0000x reference)
import functools
from typing import Any

import jax
import jax.numpy as jnp
import numpy as np
from jax.experimental import pallas as pl
from jax.experimental.pallas import tpu as pltpu

B, T, N, D = 8, 96, 207, 128
M, K, H = 1196, 8, 4
HD = D // H
TP = T + 1
OC = N * D
RO = 1656
BM = 128

_HI = jax.lax.Precision.HIGHEST


def _gelu(x):
    return x * 0.5 * (1.0 + jax.lax.erf(x * np.float32(1.0 / np.sqrt(2.0))))


def _k0_body(x_ref, w_ref, b_ref, o_ref):
    x = x_ref[...]
    xp = jnp.pad(x, ((0, 0), (0, 0), (6, 6)))
    acc = jnp.zeros((N, B, TP), jnp.float32)
    for j in range(12):
        wj = w_ref[...][:, j].reshape(N, 1, 1)
        acc = acc + wj * xp[:, :, j:j + TP]
    acc = acc + b_ref[...].reshape(N, 1, 1)
    o_ref[...] = _gelu(acc)


def _k1_body(w_ref, h_ref, e_ref, b_ref, o_ref):
    z = jnp.dot(w_ref[...], h_ref[...], precision=_HI,
                preferred_element_type=jnp.float32)
    z = _gelu(z + b_ref[...])
    o_ref[...] = jnp.dot(z, e_ref[...], precision=_HI,
                         preferred_element_type=jnp.float32)


def _k2_body(q_ref, g_ref, b_ref, qln_ref, qn_ref):
    x = q_ref[...]
    mu = jnp.mean(x, axis=-1, keepdims=True)
    xc = x - mu
    v = jnp.mean(xc * xc, axis=-1, keepdims=True)
    qln = xc * jax.lax.rsqrt(v + 1e-5) * g_ref[...].reshape(1, 1, D) \
        + b_ref[...].reshape(1, 1, D)
    qln_ref[...] = qln
    ss = jnp.sum(jnp.sum(qln * qln, axis=2), axis=1).reshape(B, 1, 1)
    qn_ref[...] = qln / jnp.maximum(jnp.sqrt(ss), 1e-12)


def _k3_body(qn_ref, mem_ref, sq_ref, ms_ref, yq_ref, my_ref, sim_ref):
    mem = mem_ref[...]
    s = jax.lax.dot_general(qn_ref[...], mem, (((1,), (1,)), ((), ())),
                            precision=_HI,
                            preferred_element_type=jnp.float32)
    nsq = jnp.sum(mem * mem, axis=1, keepdims=True)
    inv = jax.lax.rsqrt(jnp.maximum(nsq, 1e-24)).reshape(1, BM)
    s = s * inv
    mask = (sq_ref[...] == ms_ref[...]).astype(jnp.float32)
    s = s * mask + (1.0 - mask) * (-10000.0)
    dy = jnp.abs(yq_ref[...] - my_ref[...])
    s = s * (0.5 + 0.5 * (1.0 - jnp.exp(dy * (-0.5))))
    sim_ref[...] = s


def _k4_body(sim_ref, idx_ref):
    s = sim_ref[...]
    col = jax.lax.broadcasted_iota(jnp.int32, (B, M), 1)
    cols = []
    for j in range(K):
        mval = jnp.max(s, axis=1, keepdims=True)
        eq = s == mval
        ij = jnp.min(jnp.where(eq, col, jnp.int32(2**30)), axis=1)
        cols.append(ij.reshape(B, 1))
        s = jnp.where(col == ij.reshape(B, 1), -jnp.inf, s)
    idx_ref[...] = jnp.concatenate(cols, axis=1)


def _k5_body(idx_ref, mem_ref, out_ref):
    del idx_ref
    out_ref[...] = mem_ref[...]


def _k6_body(q_ref, r_ref, wq_ref, bq_ref, wk_ref, bk_ref, wv_ref, bv_ref,
             wo_ref, bo_ref, wp_ref, bp_ref, out_ref):
    qb = q_ref[...].reshape(N, D)
    r = r_ref[...].reshape(K * N, D)
    Q = jnp.dot(qb, wq_ref[...], precision=_HI,
                preferred_element_type=jnp.float32) + bq_ref[...]
    Kk = jnp.dot(r, wk_ref[...], precision=_HI,
                 preferred_element_type=jnp.float32) + bk_ref[...]
    V = jnp.dot(r, wv_ref[...], precision=_HI,
                preferred_element_type=jnp.float32) + bv_ref[...]
    scale = np.float32(1.0 / np.sqrt(HD))
    o_heads = []
    for h in range(H):
        d0 = h * HD
        Qh = Q[:, d0:d0 + HD]
        sc = []
        for k in range(K):
            Kh = Kk[k * N:(k + 1) * N, d0:d0 + HD]
            sc.append(jnp.sum(Qh * Kh, axis=1, keepdims=True) * scale)
        sc = jnp.concatenate(sc, axis=1)
        m = jnp.max(sc, axis=1, keepdims=True)
        e = jnp.exp(sc - m)
        att = e / jnp.sum(e, axis=1, keepdims=True)
        oh = jnp.zeros((N, HD), jnp.float32)
        for k in range(K):
            Vh = V[k * N:(k + 1) * N, d0:d0 + HD]
            oh = oh + att[:, k:k + 1] * Vh
        o_heads.append(oh)
    o = jnp.concatenate(o_heads, axis=1)
    o = jnp.dot(o, wo_ref[...], precision=_HI,
                preferred_element_type=jnp.float32) + bo_ref[...]
    o = jnp.dot(o, wp_ref[...], precision=_HI,
                preferred_element_type=jnp.float32) + bp_ref[...]
    out_ref[...] = o.reshape(1, N, D)


def kernel(x_scalar, season_q, year_q, dw_w, dw_b, pw_w, pw_b, ln_g, ln_b,
           wq, bq, wk, bk, wv, bv, wo, bo, proj_w, proj_b,
           mem_bank, mem_seasons, mem_years):
    f32 = jnp.float32

    x_nbt = jnp.transpose(x_scalar, (2, 0, 1))
    h = pl.pallas_call(
        _k0_body,
        out_shape=jax.ShapeDtypeStruct((N, B, TP), f32),
    )(x_nbt, dw_w.reshape(N, 12), dw_b)

    h2 = h.reshape(N, B * TP)
    w2 = pw_w.reshape(OC, N)
    emat = jnp.kron(jnp.eye(B, dtype=f32), jnp.full((TP, 1), 1.0 / TP, f32))
    q_raw = pl.pallas_call(
        _k1_body,
        grid=(OC // RO,),
        in_specs=[
            pl.BlockSpec((RO, N), lambda i: (i, 0)),
            pl.BlockSpec((N, B * TP), lambda i: (0, 0)),
            pl.BlockSpec((B * TP, B), lambda i: (0, 0)),
            pl.BlockSpec((RO, 1), lambda i: (i, 0)),
        ],
        out_specs=pl.BlockSpec((RO, B), lambda i: (i, 0)),
        out_shape=jax.ShapeDtypeStruct((OC, B), f32),
    )(w2, h2, emat, pw_b.reshape(OC, 1))

    q3 = jnp.transpose(q_raw, (1, 0)).reshape(B, N, D)
    qln, qn = pl.pallas_call(
        _k2_body,
        out_shape=(jax.ShapeDtypeStruct((B, N, D), f32),
                   jax.ShapeDtypeStruct((B, N, D), f32)),
    )(q3, ln_g.reshape(1, D), ln_b.reshape(1, D))

    memf = mem_bank.reshape(M, OC)
    nblk = (M + BM - 1) // BM
    sim = pl.pallas_call(
        _k3_body,
        grid=(nblk,),
        in_specs=[
            pl.BlockSpec((B, OC), lambda i: (0, 0)),
            pl.BlockSpec((BM, OC), lambda i: (i, 0)),
            pl.BlockSpec((B, 1), lambda i: (0, 0)),
            pl.BlockSpec((1, BM), lambda i: (0, i)),
            pl.BlockSpec((B, 1), lambda i: (0, 0)),
            pl.BlockSpec((1, BM), lambda i: (0, i)),
        ],
        out_specs=pl.BlockSpec((B, BM), lambda i: (0, i)),
        out_shape=jax.ShapeDtypeStruct((B, M), f32),
    )(qn.reshape(B, OC), memf,
      season_q.astype(jnp.int32).reshape(B, 1),
      mem_seasons.astype(jnp.int32).reshape(1, M),
      year_q.reshape(B, 1), mem_years.reshape(1, M))

    idx = pl.pallas_call(
        _k4_body,
        out_shape=jax.ShapeDtypeStruct((B, 8), jnp.int32),
    )(sim)

    idx_flat = idx.reshape(B * K)
    retrieved = pl.pallas_call(
        _k5_body,
        grid_spec=pltpu.PrefetchScalarGridSpec(
            num_scalar_prefetch=1,
            grid=(B * K,),
            in_specs=[pl.BlockSpec((1, N, D), lambda i, idx_ref: (idx_ref[i], 0, 0))],
            out_specs=pl.BlockSpec((1, N, D), lambda i, idx_ref: (i, 0, 0)),
        ),
        out_shape=jax.ShapeDtypeStruct((B * K, N, D), f32),
    )(idx_flat, mem_bank)

    r4 = retrieved.reshape(B, K, N, D)
    out = pl.pallas_call(
        _k6_body,
        grid=(B,),
        in_specs=[
            pl.BlockSpec((1, N, D), lambda i: (i, 0, 0)),
            pl.BlockSpec((1, K, N, D), lambda i: (i, 0, 0, 0)),
            pl.BlockSpec((D, D), lambda i: (0, 0)),
            pl.BlockSpec((1, D), lambda i: (0, 0)),
            pl.BlockSpec((D, D), lambda i: (0, 0)),
            pl.BlockSpec((1, D), lambda i: (0, 0)),
            pl.BlockSpec((D, D), lambda i: (0, 0)),
            pl.BlockSpec((1, D), lambda i: (0, 0)),
            pl.BlockSpec((D, D), lambda i: (0, 0)),
            pl.BlockSpec((1, D), lambda i: (0, 0)),
            pl.BlockSpec((D, D), lambda i: (0, 0)),
            pl.BlockSpec((1, D), lambda i: (0, 0)),
        ],
        out_specs=pl.BlockSpec((1, N, D), lambda i: (i, 0, 0)),
        out_shape=jax.ShapeDtypeStruct((B, N, D), f32),
    )(qln, r4,
      wq.T, bq.reshape(1, D), wk.T, bk.reshape(1, D),
      wv.T, bv.reshape(1, D), wo.T, bo.reshape(1, D),
      proj_w.T, proj_b.reshape(1, D))

    return (out, qln)


# speedup vs baseline: 1.2589x; 1.2589x over previous
import functools
from typing import Any

import jax
import jax.numpy as jnp
import numpy as np
from jax.experimental import pallas as pl
from jax.experimental.pallas import tpu as pltpu

B, T, N, D = 8, 96, 207, 128
M, K, H = 1196, 8, 4
HD = D // H
TP = T + 1
OC = N * D
RO = 2944
BM = 128

_HI = jax.lax.Precision.HIGHEST


def _gelu(x):
    return x * 0.5 * (1.0 + jax.lax.erf(x * np.float32(1.0 / np.sqrt(2.0))))


def _k0_body(x_ref, w_ref, b_ref, o_ref):
    w = w_ref[...]
    bias = b_ref[...]
    for b in range(B):
        xb = x_ref[b * T:(b + 1) * T, :]
        xpb = jnp.pad(xb, ((6, 6), (0, 0)))
        acc = jnp.zeros((TP, N), jnp.float32)
        for j in range(12):
            acc = acc + w[j:j + 1, :] * xpb[j:j + TP, :]
        o_ref[b * TP:(b + 1) * TP, :] = _gelu(acc + bias)


def _k1_body(h_ref, w_ref, e_ref, b_ref, o_ref):
    z = jax.lax.dot_general(h_ref[...], w_ref[...], (((1,), (1,)), ((), ())),
                            precision=_HI,
                            preferred_element_type=jnp.float32)
    z = _gelu(z + b_ref[...].reshape(1, RO))
    o_ref[...] = jnp.dot(e_ref[...], z, precision=_HI,
                         preferred_element_type=jnp.float32)


def _k2_body(q_ref, g_ref, b_ref, qln_ref, qn_ref):
    x = q_ref[...]
    mu = jnp.mean(x, axis=-1, keepdims=True)
    xc = x - mu
    v = jnp.mean(xc * xc, axis=-1, keepdims=True)
    qln = xc * jax.lax.rsqrt(v + 1e-5) * g_ref[...].reshape(1, 1, D) \
        + b_ref[...].reshape(1, 1, D)
    qln_ref[...] = qln
    ss = jnp.sum(jnp.sum(qln * qln, axis=2), axis=1).reshape(B, 1, 1)
    qn_ref[...] = qln / jnp.maximum(jnp.sqrt(ss), 1e-12)


def _k3_body(qn_ref, mem_ref, sq_ref, ms_ref, yq_ref, my_ref, sim_ref):
    mem = mem_ref[...]
    s = jax.lax.dot_general(qn_ref[...], mem, (((1,), (1,)), ((), ())),
                            precision=_HI,
                            preferred_element_type=jnp.float32)
    nsq = jnp.sum(mem * mem, axis=1, keepdims=True)
    inv = jax.lax.rsqrt(jnp.maximum(nsq, 1e-24)).reshape(1, BM)
    s = s * inv
    mask = (sq_ref[...] == ms_ref[...]).astype(jnp.float32)
    s = s * mask + (1.0 - mask) * (-10000.0)
    dy = jnp.abs(yq_ref[...] - my_ref[...])
    s = s * (0.5 + 0.5 * (1.0 - jnp.exp(dy * (-0.5))))
    sim_ref[...] = s


def _k4_body(sim_ref, idx_ref):
    s = sim_ref[...]
    col = jax.lax.broadcasted_iota(jnp.int32, (B, M), 1)
    cols = []
    for j in range(K):
        mval = jnp.max(s, axis=1, keepdims=True)
        eq = s == mval
        ij = jnp.min(jnp.where(eq, col, jnp.int32(2**30)), axis=1)
        cols.append(ij.reshape(B, 1))
        s = jnp.where(col == ij.reshape(B, 1), -jnp.inf, s)
    idx_ref[...] = jnp.concatenate(cols, axis=1)


def _k5_body(idx_ref, mem_ref, out_ref):
    del idx_ref
    out_ref[...] = mem_ref[...]


def _k6_body(q_ref, r_ref, wq_ref, bq_ref, wk_ref, bk_ref, wv_ref, bv_ref,
             wo_ref, bo_ref, wp_ref, bp_ref, out_ref):
    def mmT(x, w_ref, b_ref):
        return jax.lax.dot_general(
            x, w_ref[...], (((1,), (1,)), ((), ())), precision=_HI,
            preferred_element_type=jnp.float32) + b_ref[...]

    qb = q_ref[...].reshape(N, D)
    r = r_ref[...].reshape(K * N, D)
    Q = mmT(qb, wq_ref, bq_ref)
    Kk = mmT(r, wk_ref, bk_ref)
    V = mmT(r, wv_ref, bv_ref)
    scale = np.float32(1.0 / np.sqrt(HD))
    o_heads = []
    for h in range(H):
        d0 = h * HD
        Qh = Q[:, d0:d0 + HD]
        sc = []
        for k in range(K):
            Kh = Kk[k * N:(k + 1) * N, d0:d0 + HD]
            sc.append(jnp.sum(Qh * Kh, axis=1, keepdims=True) * scale)
        sc = jnp.concatenate(sc, axis=1)
        m = jnp.max(sc, axis=1, keepdims=True)
        e = jnp.exp(sc - m)
        att = e / jnp.sum(e, axis=1, keepdims=True)
        oh = jnp.zeros((N, HD), jnp.float32)
        for k in range(K):
            Vh = V[k * N:(k + 1) * N, d0:d0 + HD]
            oh = oh + att[:, k:k + 1] * Vh
        o_heads.append(oh)
    o = jnp.concatenate(o_heads, axis=1)
    o = mmT(o, wo_ref, bo_ref)
    o = mmT(o, wp_ref, bp_ref)
    out_ref[...] = o.reshape(1, N, D)


def kernel(x_scalar, season_q, year_q, dw_w, dw_b, pw_w, pw_b, ln_g, ln_b,
           wq, bq, wk, bk, wv, bv, wo, bo, proj_w, proj_b,
           mem_bank, mem_seasons, mem_years):
    f32 = jnp.float32

    x_btn = x_scalar.reshape(B * T, N)
    dw_w2 = jnp.transpose(dw_w.reshape(N, 12), (1, 0))
    h = pl.pallas_call(
        _k0_body,
        out_shape=jax.ShapeDtypeStruct((B * TP, N), f32),
    )(x_btn, dw_w2, dw_b.reshape(1, N))

    w2 = pw_w.reshape(OC, N)
    emat = jnp.asarray(
        np.kron(np.eye(B, dtype=np.float32),
                np.full((1, TP), 1.0 / TP, np.float32)))
    q_raw = pl.pallas_call(
        _k1_body,
        grid=(OC // RO,),
        in_specs=[
            pl.BlockSpec((B * TP, N), lambda i: (0, 0)),
            pl.BlockSpec((RO, N), lambda i: (i, 0)),
            pl.BlockSpec((B, B * TP), lambda i: (0, 0)),
            pl.BlockSpec((1, 1, RO), lambda i: (i, 0, 0)),
        ],
        out_specs=pl.BlockSpec((B, RO), lambda i: (0, i)),
        out_shape=jax.ShapeDtypeStruct((B, OC), f32),
    )(h, w2, emat, pw_b.reshape(OC // RO, 1, RO))

    q3 = q_raw.reshape(B, N, D)
    qln, qn = pl.pallas_call(
        _k2_body,
        out_shape=(jax.ShapeDtypeStruct((B, N, D), f32),
                   jax.ShapeDtypeStruct((B, N, D), f32)),
    )(q3, ln_g.reshape(1, D), ln_b.reshape(1, D))

    memf = mem_bank.reshape(M, OC)
    nblk = (M + BM - 1) // BM
    sim = pl.pallas_call(
        _k3_body,
        grid=(nblk,),
        in_specs=[
            pl.BlockSpec((B, OC), lambda i: (0, 0)),
            pl.BlockSpec((BM, OC), lambda i: (i, 0)),
            pl.BlockSpec((B, 1), lambda i: (0, 0)),
            pl.BlockSpec((1, BM), lambda i: (0, i)),
            pl.BlockSpec((B, 1), lambda i: (0, 0)),
            pl.BlockSpec((1, BM), lambda i: (0, i)),
        ],
        out_specs=pl.BlockSpec((B, BM), lambda i: (0, i)),
        out_shape=jax.ShapeDtypeStruct((B, M), f32),
    )(qn.reshape(B, OC), memf,
      season_q.astype(jnp.int32).reshape(B, 1),
      mem_seasons.astype(jnp.int32).reshape(1, M),
      year_q.reshape(B, 1), mem_years.reshape(1, M))

    idx = pl.pallas_call(
        _k4_body,
        out_shape=jax.ShapeDtypeStruct((B, 8), jnp.int32),
    )(sim)

    idx_flat = idx.reshape(B * K)
    retrieved = pl.pallas_call(
        _k5_body,
        grid_spec=pltpu.PrefetchScalarGridSpec(
            num_scalar_prefetch=1,
            grid=(B * K,),
            in_specs=[pl.BlockSpec((1, N, D), lambda i, idx_ref: (idx_ref[i], 0, 0))],
            out_specs=pl.BlockSpec((1, N, D), lambda i, idx_ref: (i, 0, 0)),
        ),
        out_shape=jax.ShapeDtypeStruct((B * K, N, D), f32),
    )(idx_flat, mem_bank)

    r4 = retrieved.reshape(B, K, N, D)
    out = pl.pallas_call(
        _k6_body,
        grid=(B,),
        in_specs=[
            pl.BlockSpec((1, N, D), lambda i: (i, 0, 0)),
            pl.BlockSpec((1, K, N, D), lambda i: (i, 0, 0, 0)),
            pl.BlockSpec((D, D), lambda i: (0, 0)),
            pl.BlockSpec((1, D), lambda i: (0, 0)),
            pl.BlockSpec((D, D), lambda i: (0, 0)),
            pl.BlockSpec((1, D), lambda i: (0, 0)),
            pl.BlockSpec((D, D), lambda i: (0, 0)),
            pl.BlockSpec((1, D), lambda i: (0, 0)),
            pl.BlockSpec((D, D), lambda i: (0, 0)),
            pl.BlockSpec((1, D), lambda i: (0, 0)),
            pl.BlockSpec((D, D), lambda i: (0, 0)),
            pl.BlockSpec((1, D), lambda i: (0, 0)),
        ],
        out_specs=pl.BlockSpec((1, N, D), lambda i: (i, 0, 0)),
        out_shape=jax.ShapeDtypeStruct((B, N, D), f32),
    )(qln, r4,
      wq, bq.reshape(1, D), wk, bk.reshape(1, D),
      wv, bv.reshape(1, D), wo, bo.reshape(1, D),
      proj_w, proj_b.reshape(1, D))

    return (out, qln)


# speedup vs baseline: 1.8322x; 1.4554x over previous
import functools
from typing import Any

import jax
import jax.numpy as jnp
import numpy as np
from jax.experimental import pallas as pl
from jax.experimental.pallas import tpu as pltpu

B, T, N, D = 8, 96, 207, 128
M, K, H = 1196, 8, 4
HD = D // H
TP = T + 1
OC = N * D
RO = 2944
BM = 128

_HI = jax.lax.Precision.HIGHEST


def _gelu(x):
    return x * 0.5 * (1.0 + jax.lax.erf(x * np.float32(1.0 / np.sqrt(2.0))))


def _k0_body(x_ref, w_ref, b_ref, o_ref):
    w = w_ref[...]
    bias = b_ref[...]
    for b in range(B):
        xb = x_ref[b * T:(b + 1) * T, :]
        xpb = jnp.pad(xb, ((6, 6), (0, 0)))
        acc = jnp.zeros((TP, N), jnp.float32)
        for j in range(12):
            acc = acc + w[j:j + 1, :] * xpb[j:j + TP, :]
        o_ref[b * TP:(b + 1) * TP, :] = _gelu(acc + bias).astype(jnp.bfloat16)


def _k1_body(h_ref, w_ref, b_ref, o_ref):
    z = jax.lax.dot_general(h_ref[...], w_ref[...].astype(jnp.bfloat16),
                            (((1,), (1,)), ((), ())),
                            preferred_element_type=jnp.float32)
    z = _gelu(z + b_ref[...].reshape(1, RO))
    o_ref[...] = jnp.sum(z.reshape(B, TP, RO), axis=1) * np.float32(1.0 / TP)


def _k2_body(q_ref, g_ref, b_ref, qln_ref, qn_ref):
    x = q_ref[...]
    mu = jnp.mean(x, axis=-1, keepdims=True)
    xc = x - mu
    v = jnp.mean(xc * xc, axis=-1, keepdims=True)
    qln = xc * jax.lax.rsqrt(v + 1e-5) * g_ref[...].reshape(1, 1, D) \
        + b_ref[...].reshape(1, 1, D)
    qln_ref[...] = qln
    ss = jnp.sum(jnp.sum(qln * qln, axis=2), axis=1).reshape(B, 1, 1)
    qn_ref[...] = (qln / jnp.maximum(jnp.sqrt(ss), 1e-12)).astype(jnp.bfloat16)


def _k3_body(qn_ref, mem_ref, sq_ref, ms_ref, yq_ref, my_ref, sim_ref):
    mem = mem_ref[...]
    nsq = jnp.sum(mem * mem, axis=1, keepdims=True)
    inv = 1.0 / jnp.maximum(jnp.sqrt(nsq), 1e-12)
    mn = (mem * inv).astype(jnp.bfloat16)
    s = jax.lax.dot_general(qn_ref[...], mn, (((1,), (1,)), ((), ())),
                            preferred_element_type=jnp.float32)
    mask = (sq_ref[...] == ms_ref[...]).astype(jnp.float32)
    s = s * mask + (1.0 - mask) * (-10000.0)
    dy = jnp.abs(yq_ref[...] - my_ref[...])
    s = s * (0.5 + 0.5 * (1.0 - jnp.exp(dy * (-0.5))))
    sim_ref[...] = s


def _k4_body(sim_ref, idx_ref):
    s = sim_ref[...]
    col = jax.lax.broadcasted_iota(jnp.int32, (B, M), 1)
    cols = []
    for j in range(K):
        mval = jnp.max(s, axis=1, keepdims=True)
        eq = s == mval
        ij = jnp.min(jnp.where(eq, col, jnp.int32(2**30)), axis=1)
        cols.append(ij.reshape(B, 1))
        s = jnp.where(col == ij.reshape(B, 1), -jnp.inf, s)
    idx_ref[...] = jnp.concatenate(cols, axis=1)


def _k5_body(idx_ref, mem_ref, out_ref):
    del idx_ref
    out_ref[...] = mem_ref[...]


def _k6_body(q_ref, r_ref, eh_ref, et_ref, wq_ref, bq_ref, wk_ref, bk_ref,
             wv_ref, bv_ref, wo_ref, bo_ref, wp_ref, bp_ref, out_ref):
    def mmT(x, w_ref, b_ref):
        return jax.lax.dot_general(
            x, w_ref[...], (((1,), (1,)), ((), ())),
            preferred_element_type=jnp.float32) + b_ref[...]

    qb = q_ref[...].reshape(N, D)
    Q = mmT(qb, wq_ref, bq_ref)
    eh = eh_ref[...]
    et = et_ref[...]
    Ks, Vs, sc = [], [], []
    for k in range(K):
        rk = r_ref[0, k]
        kk = mmT(rk, wk_ref, bk_ref)
        Vs.append(mmT(rk, wv_ref, bv_ref))
        sc.append(jnp.dot(Q * kk, eh,
                          preferred_element_type=jnp.float32).reshape(N, 1, H))
    s = jnp.concatenate(sc, axis=1)
    m = jnp.max(s, axis=1, keepdims=True)
    e = jnp.exp(s - m)
    att = e / jnp.sum(e, axis=1, keepdims=True)
    o = jnp.zeros((N, D), jnp.float32)
    for k in range(K):
        ab = jnp.dot(att[:, k, :], et,
                     preferred_element_type=jnp.float32)
        o = o + ab * Vs[k]
    o = mmT(o, wo_ref, bo_ref)
    o = mmT(o, wp_ref, bp_ref)
    out_ref[...] = o.reshape(1, N, D)


def kernel(x_scalar, season_q, year_q, dw_w, dw_b, pw_w, pw_b, ln_g, ln_b,
           wq, bq, wk, bk, wv, bv, wo, bo, proj_w, proj_b,
           mem_bank, mem_seasons, mem_years):
    f32 = jnp.float32

    x_btn = x_scalar.reshape(B * T, N)
    dw_w2 = jnp.transpose(dw_w.reshape(N, 12), (1, 0))
    h = pl.pallas_call(
        _k0_body,
        out_shape=jax.ShapeDtypeStruct((B * TP, N), jnp.bfloat16),
    )(x_btn, dw_w2, dw_b.reshape(1, N))

    w2 = pw_w.reshape(OC, N)
    q_raw = pl.pallas_call(
        _k1_body,
        grid=(OC // RO,),
        in_specs=[
            pl.BlockSpec((B * TP, N), lambda i: (0, 0)),
            pl.BlockSpec((RO, N), lambda i: (i, 0)),
            pl.BlockSpec((1, 1, RO), lambda i: (i, 0, 0)),
        ],
        out_specs=pl.BlockSpec((B, RO), lambda i: (0, i)),
        out_shape=jax.ShapeDtypeStruct((B, OC), f32),
    )(h, w2, pw_b.reshape(OC // RO, 1, RO))

    q3 = q_raw.reshape(B, N, D)
    qln, qn = pl.pallas_call(
        _k2_body,
        out_shape=(jax.ShapeDtypeStruct((B, N, D), f32),
                   jax.ShapeDtypeStruct((B, N, D), jnp.bfloat16)),
    )(q3, ln_g.reshape(1, D), ln_b.reshape(1, D))

    memf = mem_bank.reshape(M, OC)
    nblk = (M + BM - 1) // BM
    sim = pl.pallas_call(
        _k3_body,
        grid=(nblk,),
        in_specs=[
            pl.BlockSpec((B, OC), lambda i: (0, 0)),
            pl.BlockSpec((BM, OC), lambda i: (i, 0)),
            pl.BlockSpec((B, 1), lambda i: (0, 0)),
            pl.BlockSpec((1, BM), lambda i: (0, i)),
            pl.BlockSpec((B, 1), lambda i: (0, 0)),
            pl.BlockSpec((1, BM), lambda i: (0, i)),
        ],
        out_specs=pl.BlockSpec((B, BM), lambda i: (0, i)),
        out_shape=jax.ShapeDtypeStruct((B, M), f32),
    )(qn.reshape(B, OC), memf,
      season_q.astype(jnp.int32).reshape(B, 1),
      mem_seasons.astype(jnp.int32).reshape(1, M),
      year_q.reshape(B, 1), mem_years.reshape(1, M))

    idx = pl.pallas_call(
        _k4_body,
        out_shape=jax.ShapeDtypeStruct((B, 8), jnp.int32),
    )(sim)

    idx_flat = idx.reshape(B * K)
    retrieved = pl.pallas_call(
        _k5_body,
        grid_spec=pltpu.PrefetchScalarGridSpec(
            num_scalar_prefetch=1,
            grid=(B * K,),
            in_specs=[pl.BlockSpec((1, N, D), lambda i, idx_ref: (idx_ref[i], 0, 0))],
            out_specs=pl.BlockSpec((1, N, D), lambda i, idx_ref: (i, 0, 0)),
        ),
        out_shape=jax.ShapeDtypeStruct((B * K, N, D), f32),
    )(idx_flat, mem_bank)

    r4 = retrieved.reshape(B, K, N, D)
    eh_np = np.zeros((D, H), np.float32)
    for hh in range(H):
        eh_np[hh * HD:(hh + 1) * HD, hh] = 1.0 / np.sqrt(HD)
    eh = jnp.asarray(eh_np)
    et = jnp.asarray((eh_np.T > 0).astype(np.float32))
    out = pl.pallas_call(
        _k6_body,
        grid=(B,),
        in_specs=[
            pl.BlockSpec((1, N, D), lambda i: (i, 0, 0)),
            pl.BlockSpec((1, K, N, D), lambda i: (i, 0, 0, 0)),
            pl.BlockSpec((D, H), lambda i: (0, 0)),
            pl.BlockSpec((H, D), lambda i: (0, 0)),
            pl.BlockSpec((D, D), lambda i: (0, 0)),
            pl.BlockSpec((1, D), lambda i: (0, 0)),
            pl.BlockSpec((D, D), lambda i: (0, 0)),
            pl.BlockSpec((1, D), lambda i: (0, 0)),
            pl.BlockSpec((D, D), lambda i: (0, 0)),
            pl.BlockSpec((1, D), lambda i: (0, 0)),
            pl.BlockSpec((D, D), lambda i: (0, 0)),
            pl.BlockSpec((1, D), lambda i: (0, 0)),
            pl.BlockSpec((D, D), lambda i: (0, 0)),
            pl.BlockSpec((1, D), lambda i: (0, 0)),
        ],
        out_specs=pl.BlockSpec((1, N, D), lambda i: (i, 0, 0)),
        out_shape=jax.ShapeDtypeStruct((B, N, D), f32),
    )(qln, r4, eh, et,
      wq, bq.reshape(1, D), wk, bk.reshape(1, D),
      wv, bv.reshape(1, D), wo, bo.reshape(1, D),
      proj_w, proj_b.reshape(1, D))

    return (out, qln)


# speedup vs baseline: 2.3433x; 1.2789x over previous
import functools
from typing import Any

import jax
import jax.numpy as jnp
import numpy as np
from jax.experimental import pallas as pl
from jax.experimental.pallas import tpu as pltpu

B, T, N, D = 8, 96, 207, 128
M, K, H = 1196, 8, 4
HD = D // H
TP = T + 1
OC = N * D
RO = 2944
BM = 128

_HI = jax.lax.Precision.HIGHEST


def _gelu(x):
    return x * 0.5 * (1.0 + jax.lax.erf(x * np.float32(1.0 / np.sqrt(2.0))))


def _k0_body(x_ref, w_ref, b_ref, o_ref):
    w = w_ref[...]
    bias = b_ref[...]
    for b in range(B):
        xb = x_ref[b * T:(b + 1) * T, :]
        xpb = jnp.pad(xb, ((6, 6), (0, 0)))
        acc = jnp.zeros((TP, N), jnp.float32)
        for j in range(12):
            acc = acc + w[j:j + 1, :] * xpb[j:j + TP, :]
        o_ref[b * TP:(b + 1) * TP, :] = _gelu(acc + bias).astype(jnp.bfloat16)


def _k1_body(h_ref, w_ref, b_ref, o_ref):
    z = jax.lax.dot_general(h_ref[...], w_ref[...].astype(jnp.bfloat16),
                            (((1,), (1,)), ((), ())),
                            preferred_element_type=jnp.float32)
    z = _gelu(z + b_ref[...].reshape(1, RO))
    o_ref[...] = jnp.sum(z.reshape(B, TP, RO), axis=1) * np.float32(1.0 / TP)


def _k2_body(q_ref, g_ref, b_ref, qln_ref, qn_ref):
    x = q_ref[...]
    mu = jnp.mean(x, axis=-1, keepdims=True)
    xc = x - mu
    v = jnp.mean(xc * xc, axis=-1, keepdims=True)
    qln = xc * jax.lax.rsqrt(v + 1e-5) * g_ref[...].reshape(1, 1, D) \
        + b_ref[...].reshape(1, 1, D)
    qln_ref[...] = qln
    ss = jnp.sum(jnp.sum(qln * qln, axis=2), axis=1).reshape(B, 1, 1)
    qn_ref[...] = (qln / jnp.maximum(jnp.sqrt(ss), 1e-12)).astype(jnp.bfloat16)


def _k3_body(qn_ref, mem_ref, sq_ref, ms_ref, yq_ref, my_ref, sim_ref):
    mem = mem_ref[...]
    nsq = jnp.sum(jnp.sum(mem * mem, axis=2), axis=1)
    inv = (1.0 / jnp.maximum(jnp.sqrt(nsq), 1e-12)).reshape(BM, 1, 1)
    mn = (mem * inv).astype(jnp.bfloat16).reshape(BM, N * D)
    s = jax.lax.dot_general(qn_ref[...].reshape(B, N * D), mn,
                            (((1,), (1,)), ((), ())),
                            preferred_element_type=jnp.float32)
    mask = (sq_ref[...] == ms_ref[...]).astype(jnp.float32)
    s = s * mask + (1.0 - mask) * (-10000.0)
    dy = jnp.abs(yq_ref[...] - my_ref[...])
    s = s * (0.5 + 0.5 * (1.0 - jnp.exp(dy * (-0.5))))
    sim_ref[...] = s


def _k4_body(sim_ref, idx_ref):
    s = sim_ref[...]
    col = jax.lax.broadcasted_iota(jnp.int32, (B, M), 1)
    cols = []
    for j in range(K):
        mval = jnp.max(s, axis=1, keepdims=True)
        eq = s == mval
        ij = jnp.min(jnp.where(eq, col, jnp.int32(2**30)), axis=1)
        cols.append(ij.reshape(B, 1))
        s = jnp.where(col == ij.reshape(B, 1), -jnp.inf, s)
    idx_ref[...] = jnp.concatenate(cols, axis=1)


def _k5_body(idx_ref, mem_ref, out_ref):
    del idx_ref
    out_ref[...] = mem_ref[...]


def _k6_body(q_ref, r_ref, eh_ref, et_ref, wq_ref, bq_ref, wk_ref, bk_ref,
             wv_ref, bv_ref, wo_ref, bo_ref, wp_ref, bp_ref, out_ref):
    def mmT(x, w_ref, b_ref):
        return jax.lax.dot_general(
            x, w_ref[...], (((1,), (1,)), ((), ())),
            preferred_element_type=jnp.float32) + b_ref[...]

    qb = q_ref[...].reshape(N, D)
    Q = mmT(qb, wq_ref, bq_ref)
    eh = eh_ref[...]
    et = et_ref[...]
    Ks, Vs, sc = [], [], []
    for k in range(K):
        rk = r_ref[0, k]
        kk = mmT(rk, wk_ref, bk_ref)
        Vs.append(mmT(rk, wv_ref, bv_ref))
        sc.append(jnp.dot(Q * kk, eh,
                          preferred_element_type=jnp.float32).reshape(N, 1, H))
    s = jnp.concatenate(sc, axis=1)
    m = jnp.max(s, axis=1, keepdims=True)
    e = jnp.exp(s - m)
    att = e / jnp.sum(e, axis=1, keepdims=True)
    o = jnp.zeros((N, D), jnp.float32)
    for k in range(K):
        ab = jnp.dot(att[:, k, :], et,
                     preferred_element_type=jnp.float32)
        o = o + ab * Vs[k]
    o = mmT(o, wo_ref, bo_ref)
    o = mmT(o, wp_ref, bp_ref)
    out_ref[...] = o.reshape(1, N, D)


def kernel(x_scalar, season_q, year_q, dw_w, dw_b, pw_w, pw_b, ln_g, ln_b,
           wq, bq, wk, bk, wv, bv, wo, bo, proj_w, proj_b,
           mem_bank, mem_seasons, mem_years):
    f32 = jnp.float32

    x_btn = x_scalar.reshape(B * T, N)
    dw_w2 = jnp.transpose(dw_w.reshape(N, 12), (1, 0))
    h = pl.pallas_call(
        _k0_body,
        out_shape=jax.ShapeDtypeStruct((B * TP, N), jnp.bfloat16),
    )(x_btn, dw_w2, dw_b.reshape(1, N))

    w2 = pw_w.reshape(OC, N)
    q_raw = pl.pallas_call(
        _k1_body,
        grid=(OC // RO,),
        in_specs=[
            pl.BlockSpec((B * TP, N), lambda i: (0, 0)),
            pl.BlockSpec((RO, N), lambda i: (i, 0)),
            pl.BlockSpec((1, 1, RO), lambda i: (i, 0, 0)),
        ],
        out_specs=pl.BlockSpec((B, RO), lambda i: (0, i)),
        out_shape=jax.ShapeDtypeStruct((B, OC), f32),
    )(h, w2, pw_b.reshape(OC // RO, 1, RO))

    q3 = q_raw.reshape(B, N, D)
    qln, qn = pl.pallas_call(
        _k2_body,
        out_shape=(jax.ShapeDtypeStruct((B, N, D), f32),
                   jax.ShapeDtypeStruct((B, N, D), jnp.bfloat16)),
    )(q3, ln_g.reshape(1, D), ln_b.reshape(1, D))

    nblk = (M + BM - 1) // BM
    sim = pl.pallas_call(
        _k3_body,
        grid=(nblk,),
        in_specs=[
            pl.BlockSpec((B, N, D), lambda i: (0, 0, 0)),
            pl.BlockSpec((BM, N, D), lambda i: (i, 0, 0)),
            pl.BlockSpec((B, 1), lambda i: (0, 0)),
            pl.BlockSpec((1, BM), lambda i: (0, i)),
            pl.BlockSpec((B, 1), lambda i: (0, 0)),
            pl.BlockSpec((1, BM), lambda i: (0, i)),
        ],
        out_specs=pl.BlockSpec((B, BM), lambda i: (0, i)),
        out_shape=jax.ShapeDtypeStruct((B, M), f32),
    )(qn, mem_bank,
      season_q.astype(jnp.int32).reshape(B, 1),
      mem_seasons.astype(jnp.int32).reshape(1, M),
      year_q.reshape(B, 1), mem_years.reshape(1, M))

    idx = pl.pallas_call(
        _k4_body,
        out_shape=jax.ShapeDtypeStruct((B, 8), jnp.int32),
    )(sim)

    idx_flat = idx.reshape(B * K)
    retrieved = pl.pallas_call(
        _k5_body,
        grid_spec=pltpu.PrefetchScalarGridSpec(
            num_scalar_prefetch=1,
            grid=(B * K,),
            in_specs=[pl.BlockSpec((1, N, D), lambda i, idx_ref: (idx_ref[i], 0, 0))],
            out_specs=pl.BlockSpec((1, N, D), lambda i, idx_ref: (i, 0, 0)),
        ),
        out_shape=jax.ShapeDtypeStruct((B * K, N, D), f32),
    )(idx_flat, mem_bank)

    r4 = retrieved.reshape(B, K, N, D)
    eh_np = np.zeros((D, H), np.float32)
    for hh in range(H):
        eh_np[hh * HD:(hh + 1) * HD, hh] = 1.0 / np.sqrt(HD)
    eh = jnp.asarray(eh_np)
    et = jnp.asarray((eh_np.T > 0).astype(np.float32))
    out = pl.pallas_call(
        _k6_body,
        grid=(B,),
        in_specs=[
            pl.BlockSpec((1, N, D), lambda i: (i, 0, 0)),
            pl.BlockSpec((1, K, N, D), lambda i: (i, 0, 0, 0)),
            pl.BlockSpec((D, H), lambda i: (0, 0)),
            pl.BlockSpec((H, D), lambda i: (0, 0)),
            pl.BlockSpec((D, D), lambda i: (0, 0)),
            pl.BlockSpec((1, D), lambda i: (0, 0)),
            pl.BlockSpec((D, D), lambda i: (0, 0)),
            pl.BlockSpec((1, D), lambda i: (0, 0)),
            pl.BlockSpec((D, D), lambda i: (0, 0)),
            pl.BlockSpec((1, D), lambda i: (0, 0)),
            pl.BlockSpec((D, D), lambda i: (0, 0)),
            pl.BlockSpec((1, D), lambda i: (0, 0)),
            pl.BlockSpec((D, D), lambda i: (0, 0)),
            pl.BlockSpec((1, D), lambda i: (0, 0)),
        ],
        out_specs=pl.BlockSpec((1, N, D), lambda i: (i, 0, 0)),
        out_shape=jax.ShapeDtypeStruct((B, N, D), f32),
    )(qln, r4, eh, et,
      wq, bq.reshape(1, D), wk, bk.reshape(1, D),
      wv, bv.reshape(1, D), wo, bo.reshape(1, D),
      proj_w, proj_b.reshape(1, D))

    return (out, qln)
